# baseline (speedup 1.0000x reference)
import functools

import jax
import jax.numpy as jnp
from jax import lax
from jax.experimental import pallas as pl
from jax.experimental.pallas import tpu as pltpu
from jax.experimental.pallas import tpu_sc as plsc

N_NODES = 10000
N_PAD = 10240
N_EDGES = 160000
D = 256
W = D // 2
L = 16
NC, NS = 2, 16
NW = NC * NS
B = 64
EPW = 5120
NCHUNK = EPW // B
E_PAD = NW * EPW


def _row_dot(rows_s, rows_d, row):
    def prod(j):
        ws = rows_s[row, pl.ds(j * L, L)]
        wd = rows_d[row, pl.ds(j * L, L)]
        return plsc.bitcast(ws, jnp.bfloat16) * plsc.bitcast(wd, jnp.bfloat16)

    half = W // L // 2
    acc_a = prod(0)
    acc_b = prod(half)
    for j in range(1, half):
        acc_a = acc_a + prod(j)
        acc_b = acc_b + prod(half + j)
    u, v = plsc.unpack(acc_a + acc_b, format=plsc.PackFormat.INTERLEAVED)
    return u + v


def _body(h_hbm, src_hbm, dst_hbm, out_hbm,
          h_sp, is0, id0, is1, id1, rs0, rd0, rs1, rd1, scores,
          sem_i0, sem_i1, sem_s0, sem_d0, sem_s1, sem_d1):
    sid = lax.axis_index("s")
    wid = sid * NC + lax.axis_index("c")
    rows_per_sub = N_PAD // NS
    pltpu.sync_copy(h_hbm.at[pl.ds(sid * rows_per_sub, rows_per_sub)],
                    h_sp.at[pl.ds(sid * rows_per_sub, rows_per_sub)])
    plsc.subcore_barrier()

    row_ids = lax.iota(jnp.int32, L)
    bufs = ((is0, id0, sem_i0, rs0, rd0, sem_s0, sem_d0),
            (is1, id1, sem_i1, rs1, rd1, sem_s1, sem_d1))

    def fetch_idx(c, buf):
        i_s, i_d, si = bufs[buf][:3]
        pltpu.make_async_copy(src_hbm.at[wid, c], i_s, si).start()
        pltpu.make_async_copy(dst_hbm.at[wid, c], i_d, si).start()

    def wait_idx(c, buf):
        i_s, i_d, si = bufs[buf][:3]
        pltpu.make_async_copy(src_hbm.at[wid, c], i_s, si).wait()
        pltpu.make_async_copy(dst_hbm.at[wid, c], i_d, si).wait()

    def issue_rows(buf, c):
        i_s, i_d, _, rs, rd, ss, sd = bufs[buf]
        pltpu.make_async_copy(h_sp.at[i_s], rs, ss).start()
        pltpu.make_async_copy(h_sp.at[i_d], rd, sd).start()

    def wait_rows(buf):
        i_s, i_d, _, rs, rd, ss, sd = bufs[buf]
        pltpu.make_async_copy(h_sp.at[i_s], rs, ss).wait()
        pltpu.make_async_copy(h_sp.at[i_d], rd, sd).wait()

    def compute(c, buf):
        rs, rd = bufs[buf][3], bufs[buf][4]

        def group(g, _):
            base = g * L
            sv = jnp.zeros((L,), jnp.float32)
            for r in range(L):
                acc = _row_dot(rs, rd, base + r)
                sv = jnp.where(row_ids == r, jnp.sum(acc), sv)
            scores[pl.ds(c * B + base, L)] = sv
            return ()

        lax.fori_loop(0, B // L, group, ())

    fetch_idx(0, 0)
    fetch_idx(1, 1)
    wait_idx(0, 0)
    issue_rows(0, 0)

    def pair(p, _):
        c0 = 2 * p
        wait_idx(c0 + 1, 1)
        issue_rows(1, c0 + 1)

        @pl.when(c0 + 2 < NCHUNK)
        def _():
            fetch_idx(c0 + 2, 0)

        wait_rows(0)
        compute(c0, 0)

        @pl.when(c0 + 2 < NCHUNK)
        def _():
            wait_idx(c0 + 2, 0)
            issue_rows(0, c0 + 2)

        @pl.when(c0 + 3 < NCHUNK)
        def _():
            fetch_idx(c0 + 3, 1)

        wait_rows(1)
        compute(c0 + 1, 1)
        return ()

    lax.fori_loop(0, NCHUNK // 2, pair, ())
    pltpu.sync_copy(scores, out_hbm.at[pl.ds(wid * EPW, EPW)])


@jax.jit
def _run(h_words, src, dst):
    mesh = plsc.VectorSubcoreMesh(core_axis_name="c", subcore_axis_name="s")
    k = functools.partial(
        pl.kernel,
        out_type=jax.ShapeDtypeStruct((E_PAD,), jnp.float32),
        mesh=mesh,
        compiler_params=pltpu.CompilerParams(needs_layout_passes=False),
        scratch_types=[
            pltpu.VMEM_SHARED((N_PAD, W), jnp.int32),
            pltpu.VMEM((B,), jnp.int32),
            pltpu.VMEM((B,), jnp.int32),
            pltpu.VMEM((B,), jnp.int32),
            pltpu.VMEM((B,), jnp.int32),
            pltpu.VMEM((B, W), jnp.int32),
            pltpu.VMEM((B, W), jnp.int32),
            pltpu.VMEM((B, W), jnp.int32),
            pltpu.VMEM((B, W), jnp.int32),
            pltpu.VMEM((EPW,), jnp.float32),
            pltpu.SemaphoreType.DMA,
            pltpu.SemaphoreType.DMA,
            pltpu.SemaphoreType.DMA,
            pltpu.SemaphoreType.DMA,
            pltpu.SemaphoreType.DMA,
            pltpu.SemaphoreType.DMA,
        ],
    )(_body)
    return k(h_words, src, dst)


def kernel(h, edge_index):
    h_words = lax.bitcast_convert_type(
        h.astype(jnp.bfloat16).reshape(N_NODES, W, 2), jnp.int32)
    h_words = jnp.concatenate(
        [h_words, jnp.zeros((N_PAD - N_NODES, W), jnp.int32)])
    src = edge_index[0].astype(jnp.int32)
    dst = edge_index[1].astype(jnp.int32)
    pad = E_PAD - N_EDGES
    src = jnp.concatenate([src, jnp.zeros((pad,), jnp.int32)])
    dst = jnp.concatenate([dst, jnp.zeros((pad,), jnp.int32)])
    out = _run(h_words, src.reshape(NW, NCHUNK, B), dst.reshape(NW, NCHUNK, B))
    return out[:N_EDGES]


# speedup vs baseline: 1.6347x; 1.6347x over previous
import functools

import jax
import jax.numpy as jnp
from jax import lax
from jax.experimental import pallas as pl
from jax.experimental.pallas import tpu as pltpu
from jax.experimental.pallas import tpu_sc as plsc

N_NODES = 10000
N_PAD = 10240
N_EDGES = 160000
D = 256
W = D // 2
L = 16
NC, NS = 2, 16
NW = NC * NS
B = 64
EPW = 5120
NCHUNK = EPW // B
E_PAD = NW * EPW


def _row_dot(rows_s, rows_d, row):
    accs = []
    for j in range(W // L):
        ws = rows_s[row, pl.ds(j * L, L)]
        wd = rows_d[row, pl.ds(j * L, L)]
        prod = plsc.bitcast(ws, jnp.bfloat16) * plsc.bitcast(wd, jnp.bfloat16)
        u, v = plsc.unpack(prod, format=plsc.PackFormat.INTERLEAVED)
        accs.append(u)
        accs.append(v)
    while len(accs) > 1:
        accs = [a + b for a, b in zip(accs[::2], accs[1::2])]
    return accs[0]


def _body(h_hbm, src_hbm, dst_hbm, out_hbm,
          h_sp, is0, id0, is1, id1, rs0, rd0, rs1, rd1, scores,
          sem_i0, sem_i1, sem_s0, sem_d0, sem_s1, sem_d1):
    sid = lax.axis_index("s")
    wid = sid * NC + lax.axis_index("c")
    rows_per_sub = N_PAD // NS
    pltpu.sync_copy(h_hbm.at[pl.ds(sid * rows_per_sub, rows_per_sub)],
                    h_sp.at[pl.ds(sid * rows_per_sub, rows_per_sub)])
    plsc.subcore_barrier()

    row_ids = lax.iota(jnp.int32, L)
    bufs = ((is0, id0, sem_i0, rs0, rd0, sem_s0, sem_d0),
            (is1, id1, sem_i1, rs1, rd1, sem_s1, sem_d1))

    def fetch_idx(c, buf):
        i_s, i_d, si = bufs[buf][:3]
        pltpu.make_async_copy(src_hbm.at[wid, c], i_s, si).start()
        pltpu.make_async_copy(dst_hbm.at[wid, c], i_d, si).start()

    def wait_idx(c, buf):
        i_s, i_d, si = bufs[buf][:3]
        pltpu.make_async_copy(src_hbm.at[wid, c], i_s, si).wait()
        pltpu.make_async_copy(dst_hbm.at[wid, c], i_d, si).wait()

    def issue_rows(buf, c):
        i_s, i_d, _, rs, rd, ss, sd = bufs[buf]
        pltpu.make_async_copy(h_sp.at[i_s], rs, ss).start()
        pltpu.make_async_copy(h_sp.at[i_d], rd, sd).start()

    def wait_rows(buf):
        i_s, i_d, _, rs, rd, ss, sd = bufs[buf]
        pltpu.make_async_copy(h_sp.at[i_s], rs, ss).wait()
        pltpu.make_async_copy(h_sp.at[i_d], rd, sd).wait()

    def compute(c, buf):
        rs, rd = bufs[buf][3], bufs[buf][4]

        def group(g, _):
            base = g * L
            sv = jnp.zeros((L,), jnp.float32)
            for r in range(L):
                acc = _row_dot(rs, rd, base + r)
                sv = jnp.where(row_ids == r, jnp.sum(acc), sv)
            scores[pl.ds(c * B + base, L)] = sv
            return ()

        lax.fori_loop(0, B // L, group, ())

    fetch_idx(0, 0)
    fetch_idx(1, 1)
    wait_idx(0, 0)
    issue_rows(0, 0)

    def pair(p, _):
        c0 = 2 * p
        wait_idx(c0 + 1, 1)
        issue_rows(1, c0 + 1)

        @pl.when(c0 + 2 < NCHUNK)
        def _():
            fetch_idx(c0 + 2, 0)

        wait_rows(0)
        compute(c0, 0)

        @pl.when(c0 + 2 < NCHUNK)
        def _():
            wait_idx(c0 + 2, 0)
            issue_rows(0, c0 + 2)

        @pl.when(c0 + 3 < NCHUNK)
        def _():
            fetch_idx(c0 + 3, 1)

        wait_rows(1)
        compute(c0 + 1, 1)
        return ()

    lax.fori_loop(0, NCHUNK // 2, pair, ())
    pltpu.sync_copy(scores, out_hbm.at[pl.ds(wid * EPW, EPW)])


@jax.jit
def _run(h_words, src, dst):
    mesh = plsc.VectorSubcoreMesh(core_axis_name="c", subcore_axis_name="s")
    k = functools.partial(
        pl.kernel,
        out_type=jax.ShapeDtypeStruct((E_PAD,), jnp.float32),
        mesh=mesh,
        compiler_params=pltpu.CompilerParams(needs_layout_passes=False),
        scratch_types=[
            pltpu.VMEM_SHARED((N_PAD, W), jnp.int32),
            pltpu.VMEM((B,), jnp.int32),
            pltpu.VMEM((B,), jnp.int32),
            pltpu.VMEM((B,), jnp.int32),
            pltpu.VMEM((B,), jnp.int32),
            pltpu.VMEM((B, W), jnp.int32),
            pltpu.VMEM((B, W), jnp.int32),
            pltpu.VMEM((B, W), jnp.int32),
            pltpu.VMEM((B, W), jnp.int32),
            pltpu.VMEM((EPW,), jnp.float32),
            pltpu.SemaphoreType.DMA,
            pltpu.SemaphoreType.DMA,
            pltpu.SemaphoreType.DMA,
            pltpu.SemaphoreType.DMA,
            pltpu.SemaphoreType.DMA,
            pltpu.SemaphoreType.DMA,
        ],
    )(_body)
    return k(h_words, src, dst)


def kernel(h, edge_index):
    h_words = lax.bitcast_convert_type(
        h.astype(jnp.bfloat16).reshape(N_NODES, W, 2), jnp.int32)
    h_words = jnp.concatenate(
        [h_words, jnp.zeros((N_PAD - N_NODES, W), jnp.int32)])
    src = edge_index[0].astype(jnp.int32)
    dst = edge_index[1].astype(jnp.int32)
    pad = E_PAD - N_EDGES
    src = jnp.concatenate([src, jnp.zeros((pad,), jnp.int32)])
    dst = jnp.concatenate([dst, jnp.zeros((pad,), jnp.int32)])
    out = _run(h_words, src.reshape(NW, NCHUNK, B), dst.reshape(NW, NCHUNK, B))
    return out[:N_EDGES]


# speedup vs baseline: 1.6734x; 1.0237x over previous
import functools

import jax
import jax.numpy as jnp
from jax import lax
from jax.experimental import pallas as pl
from jax.experimental.pallas import tpu as pltpu
from jax.experimental.pallas import tpu_sc as plsc

N_NODES = 10000
N_EDGES = 160000
D = 256
W = D // 2
L = 16
NC, NS = 2, 16
NW = NC * NS
B = 64
EPW = 5120
NCHUNK = EPW // B
EPW_LAST = N_EDGES - (NW - 1) * EPW
NCHUNK_LAST = EPW_LAST // B
STAGE = 640
STAGE_LAST = N_NODES - (NS - 1) * STAGE


def _row_dot(rows_s, rows_d, row):
    accs = []
    for j in range(W // L):
        ws = rows_s[row, pl.ds(j * L, L)]
        wd = rows_d[row, pl.ds(j * L, L)]
        prod = plsc.bitcast(ws, jnp.bfloat16) * plsc.bitcast(wd, jnp.bfloat16)
        u, v = plsc.unpack(prod, format=plsc.PackFormat.INTERLEAVED)
        accs.append(u)
        accs.append(v)
    while len(accs) > 1:
        accs = [a + b for a, b in zip(accs[::2], accs[1::2])]
    return accs[0]


def _body(h_hbm, ei_hbm, out_hbm,
          h_sp, is0, id0, is1, id1, rs0, rd0, rs1, rd1, scores,
          sem_i0, sem_i1, sem_s0, sem_d0, sem_s1, sem_d1):
    sid = lax.axis_index("s")
    wid = sid * NC + lax.axis_index("c")
    @pl.when(sid < NS - 1)
    def _():
        pltpu.sync_copy(h_hbm.at[pl.ds(sid * STAGE, STAGE)],
                        h_sp.at[pl.ds(sid * STAGE, STAGE)])

    @pl.when(sid == NS - 1)
    def _():
        pltpu.sync_copy(h_hbm.at[pl.ds((NS - 1) * STAGE, STAGE_LAST)],
                        h_sp.at[pl.ds((NS - 1) * STAGE, STAGE_LAST)])

    plsc.subcore_barrier()

    ebase = wid * EPW
    nchunk = jnp.where(wid == NW - 1, NCHUNK_LAST, NCHUNK)
    row_ids = lax.iota(jnp.int32, L)
    bufs = ((is0, id0, sem_i0, rs0, rd0, sem_s0, sem_d0),
            (is1, id1, sem_i1, rs1, rd1, sem_s1, sem_d1))

    def fetch_idx(c, buf):
        i_s, i_d, si = bufs[buf][:3]
        pltpu.make_async_copy(ei_hbm.at[0, pl.ds(ebase + c * B, B)], i_s, si).start()
        pltpu.make_async_copy(ei_hbm.at[1, pl.ds(ebase + c * B, B)], i_d, si).start()

    def wait_idx(c, buf):
        i_s, i_d, si = bufs[buf][:3]
        pltpu.make_async_copy(ei_hbm.at[0, pl.ds(ebase + c * B, B)], i_s, si).wait()
        pltpu.make_async_copy(ei_hbm.at[1, pl.ds(ebase + c * B, B)], i_d, si).wait()

    def issue_rows(buf):
        i_s, i_d, _, rs, rd, ss, sd = bufs[buf]
        pltpu.make_async_copy(h_sp.at[i_s], rs, ss).start()
        pltpu.make_async_copy(h_sp.at[i_d], rd, sd).start()

    def wait_rows(buf):
        i_s, i_d, _, rs, rd, ss, sd = bufs[buf]
        pltpu.make_async_copy(h_sp.at[i_s], rs, ss).wait()
        pltpu.make_async_copy(h_sp.at[i_d], rd, sd).wait()

    def compute(c, buf):
        rs, rd = bufs[buf][3], bufs[buf][4]

        def group(g, _):
            base = g * L
            sv = jnp.zeros((L,), jnp.float32)
            for r in range(L):
                acc = _row_dot(rs, rd, base + r)
                sv = jnp.where(row_ids == r, jnp.sum(acc), sv)
            scores[pl.ds(c * B + base, L)] = sv
            return ()

        lax.fori_loop(0, B // L, group, ())

    fetch_idx(0, 0)
    fetch_idx(1, 1)
    wait_idx(0, 0)
    issue_rows(0)

    def pair(p, _):
        c0 = 2 * p
        wait_idx(c0 + 1, 1)
        issue_rows(1)

        @pl.when(c0 + 2 < nchunk)
        def _():
            fetch_idx(c0 + 2, 0)

        wait_rows(0)
        compute(c0, 0)

        @pl.when(c0 + 2 < nchunk)
        def _():
            wait_idx(c0 + 2, 0)
            issue_rows(0)

        @pl.when(c0 + 3 < nchunk)
        def _():
            fetch_idx(c0 + 3, 1)

        wait_rows(1)
        compute(c0 + 1, 1)
        return ()

    lax.fori_loop(0, nchunk // 2, pair, ())

    @pl.when(wid < NW - 1)
    def _():
        pltpu.sync_copy(scores, out_hbm.at[pl.ds(ebase, EPW)])

    @pl.when(wid == NW - 1)
    def _():
        pltpu.sync_copy(scores.at[pl.ds(0, EPW_LAST)],
                        out_hbm.at[pl.ds(ebase, EPW_LAST)])


@jax.jit
def _run(h_bf, ei):
    mesh = plsc.VectorSubcoreMesh(core_axis_name="c", subcore_axis_name="s")
    k = functools.partial(
        pl.kernel,
        out_type=jax.ShapeDtypeStruct((N_EDGES,), jnp.float32),
        mesh=mesh,
        compiler_params=pltpu.CompilerParams(needs_layout_passes=False),
        scratch_types=[
            pltpu.VMEM_SHARED((N_NODES, W), jnp.int32),
            pltpu.VMEM((B,), jnp.int32),
            pltpu.VMEM((B,), jnp.int32),
            pltpu.VMEM((B,), jnp.int32),
            pltpu.VMEM((B,), jnp.int32),
            pltpu.VMEM((B, W), jnp.int32),
            pltpu.VMEM((B, W), jnp.int32),
            pltpu.VMEM((B, W), jnp.int32),
            pltpu.VMEM((B, W), jnp.int32),
            pltpu.VMEM((EPW,), jnp.float32),
            pltpu.SemaphoreType.DMA,
            pltpu.SemaphoreType.DMA,
            pltpu.SemaphoreType.DMA,
            pltpu.SemaphoreType.DMA,
            pltpu.SemaphoreType.DMA,
            pltpu.SemaphoreType.DMA,
        ],
    )(_body)
    return k(h_bf, ei)


def kernel(h, edge_index):
    h_words = lax.bitcast_convert_type(
        h.astype(jnp.bfloat16).reshape(N_NODES, W, 2), jnp.int32)
    return _run(h_words, edge_index.astype(jnp.int32))


# speedup vs baseline: 3.4544x; 2.0642x over previous
import functools

import jax
import jax.numpy as jnp
from jax import lax
from jax.experimental import pallas as pl
from jax.experimental.pallas import tpu as pltpu
from jax.experimental.pallas import tpu_sc as plsc

N_NODES = 10000
N_EDGES = 160000
D = 256
W = D // 2
L = 16
NC, NS = 2, 16
NW = NC * NS
B = 64
EPW = 5120
NCHUNK = EPW // B
EPW_LAST = N_EDGES - (NW - 1) * EPW
NCHUNK_LAST = EPW_LAST // B
STAGE = 640
STAGE_LAST = N_NODES - (NS - 1) * STAGE


def _row_dot(rows_s, rows_d, row):
    accs = []
    for j in range(W // L):
        ws = rows_s[row, pl.ds(j * L, L)]
        wd = rows_d[row, pl.ds(j * L, L)]
        prod = plsc.bitcast(ws, jnp.bfloat16) * plsc.bitcast(wd, jnp.bfloat16)
        u, v = plsc.unpack(prod, format=plsc.PackFormat.INTERLEAVED)
        accs.append(u)
        accs.append(v)
    while len(accs) > 1:
        accs = [a + b for a, b in zip(accs[::2], accs[1::2])]
    return accs[0]


def _body(h_hbm, ei_hbm, out_hbm,
          h_sp, is0, id0, is1, id1, rs0, rd0, rs1, rd1, scores,
          sem_i0, sem_i1, sem_s0, sem_d0, sem_s1, sem_d1):
    sid = lax.axis_index("s")
    wid = sid * NC + lax.axis_index("c")
    @pl.when(sid < NS - 1)
    def _():
        pltpu.sync_copy(h_hbm.at[pl.ds(sid * STAGE, STAGE)],
                        h_sp.at[pl.ds(sid * STAGE, STAGE)])

    @pl.when(sid == NS - 1)
    def _():
        pltpu.sync_copy(h_hbm.at[pl.ds((NS - 1) * STAGE, STAGE_LAST)],
                        h_sp.at[pl.ds((NS - 1) * STAGE, STAGE_LAST)])

    plsc.subcore_barrier()

    ebase = wid * EPW
    nchunk = jnp.where(wid == NW - 1, NCHUNK_LAST, NCHUNK)
    row_ids = lax.iota(jnp.int32, L)
    bufs = ((is0, id0, sem_i0, rs0, rd0, sem_s0, sem_d0),
            (is1, id1, sem_i1, rs1, rd1, sem_s1, sem_d1))

    def fetch_idx(c, buf):
        i_s, i_d, si = bufs[buf][:3]
        pltpu.make_async_copy(ei_hbm.at[0, pl.ds(ebase + c * B, B)], i_s, si).start()
        pltpu.make_async_copy(ei_hbm.at[1, pl.ds(ebase + c * B, B)], i_d, si).start()

    def wait_idx(c, buf):
        i_s, i_d, si = bufs[buf][:3]
        pltpu.make_async_copy(ei_hbm.at[0, pl.ds(ebase + c * B, B)], i_s, si).wait()
        pltpu.make_async_copy(ei_hbm.at[1, pl.ds(ebase + c * B, B)], i_d, si).wait()

    def issue_rows(buf):
        i_s, i_d, _, rs, rd, ss, sd = bufs[buf]
        pltpu.make_async_copy(h_sp.at[i_s], rs, ss).start()
        pltpu.make_async_copy(h_sp.at[i_d], rd, sd).start()

    def wait_rows(buf):
        i_s, i_d, _, rs, rd, ss, sd = bufs[buf]
        pltpu.make_async_copy(h_sp.at[i_s], rs, ss).wait()
        pltpu.make_async_copy(h_sp.at[i_d], rd, sd).wait()

    def compute(c, buf):
        rs, rd = bufs[buf][3], bufs[buf][4]

        def group(g, _):
            base = g * L
            sv = jnp.zeros((L,), jnp.float32)
            for r in range(L):
                acc = _row_dot(rs, rd, base + r)
                sv = jnp.where(row_ids == r, jnp.sum(acc), sv)
            scores[pl.ds(c * B + base, L)] = sv
            return ()

        lax.fori_loop(0, B // L, group, ())

    fetch_idx(0, 0)
    fetch_idx(1, 1)
    wait_idx(0, 0)
    issue_rows(0)

    def pair(p, _):
        c0 = 2 * p
        wait_idx(c0 + 1, 1)
        issue_rows(1)

        @pl.when(c0 + 2 < nchunk)
        def _():
            fetch_idx(c0 + 2, 0)

        wait_rows(0)
        compute(c0, 0)

        @pl.when(c0 + 2 < nchunk)
        def _():
            wait_idx(c0 + 2, 0)
            issue_rows(0)

        @pl.when(c0 + 3 < nchunk)
        def _():
            fetch_idx(c0 + 3, 1)

        wait_rows(1)
        compute(c0 + 1, 1)
        return ()

    lax.fori_loop(0, nchunk // 2, pair, ())

    @pl.when(wid < NW - 1)
    def _():
        pltpu.sync_copy(scores, out_hbm.at[pl.ds(ebase, EPW)])

    @pl.when(wid == NW - 1)
    def _():
        pltpu.sync_copy(scores.at[pl.ds(0, EPW_LAST)],
                        out_hbm.at[pl.ds(ebase, EPW_LAST)])


@jax.jit
def _run(h_bf, ei):
    mesh = plsc.VectorSubcoreMesh(core_axis_name="c", subcore_axis_name="s")
    k = functools.partial(
        pl.kernel,
        out_type=jax.ShapeDtypeStruct((N_EDGES,), jnp.float32),
        mesh=mesh,
        compiler_params=pltpu.CompilerParams(needs_layout_passes=False),
        scratch_types=[
            pltpu.VMEM_SHARED((N_NODES, W), jnp.int32),
            pltpu.VMEM((B,), jnp.int32),
            pltpu.VMEM((B,), jnp.int32),
            pltpu.VMEM((B,), jnp.int32),
            pltpu.VMEM((B,), jnp.int32),
            pltpu.VMEM((B, W), jnp.int32),
            pltpu.VMEM((B, W), jnp.int32),
            pltpu.VMEM((B, W), jnp.int32),
            pltpu.VMEM((B, W), jnp.int32),
            pltpu.VMEM((EPW,), jnp.float32),
            pltpu.SemaphoreType.DMA,
            pltpu.SemaphoreType.DMA,
            pltpu.SemaphoreType.DMA,
            pltpu.SemaphoreType.DMA,
            pltpu.SemaphoreType.DMA,
            pltpu.SemaphoreType.DMA,
        ],
    )(_body)
    return k(h_bf, ei)


def kernel(h, edge_index):
    u = lax.bitcast_convert_type(h, jnp.uint32)
    lsb = (u >> 16) & jnp.uint32(1)
    t = (u + jnp.uint32(0x7FFF) + lsb) >> 16
    w = t[:, :W] | (t[:, W:] << 16)
    return _run(lax.bitcast_convert_type(w, jnp.int32),
                edge_index.astype(jnp.int32))
